# baseline (speedup 1.0000x reference)
import numpy as np
import jax
import jax.numpy as jnp
from jax import lax
from jax.experimental import pallas as pl
from jax.experimental.pallas import tpu as pltpu

_NB_CLASSES = 1000
_N_ROWS = 16384
_SMOOTH = np.float32(0.1)
_LOW = _SMOOTH / np.float32(_NB_CLASSES)
_HOT = (np.float32(1.0) - _SMOOTH) + _LOW

_BI = 512
_GRID = _N_ROWS // _BI


def _body(x_ref, o_ref):
    xv = x_ref[0]
    cls = lax.broadcasted_iota(jnp.int32, (_NB_CLASSES, _BI), 0)
    o_ref[...] = jnp.where(cls == xv, _HOT, _LOW)


def kernel(x):
    x3 = x.astype(jnp.int32).reshape(_GRID, 1, _BI)
    out_t = pl.pallas_call(
        _body,
        grid=(_GRID,),
        in_specs=[pl.BlockSpec((1, 1, _BI), lambda i: (i, 0, 0))],
        out_specs=pl.BlockSpec((_NB_CLASSES, _BI), lambda i: (0, i)),
        out_shape=jax.ShapeDtypeStruct((_NB_CLASSES, _N_ROWS), jnp.float32),
        compiler_params=pltpu.CompilerParams(
            dimension_semantics=("arbitrary",),
        ),
    )(x3)
    return out_t.T


# speedup vs baseline: 1.3038x; 1.3038x over previous
import numpy as np
import jax
import jax.numpy as jnp
from jax import lax
from jax.experimental import pallas as pl
from jax.experimental.pallas import tpu as pltpu

_NB_CLASSES = 1000
_N_ROWS = 16384
_SMOOTH = np.float32(0.1)
_LOW = _SMOOTH / np.float32(_NB_CLASSES)
_HOT = (np.float32(1.0) - _SMOOTH) + _LOW

_BI = 1024
_GRID = _N_ROWS // _BI


def _body(x_ref, o_ref):
    xv = x_ref[0]
    cls = lax.broadcasted_iota(jnp.int32, (_NB_CLASSES, _BI), 0)
    o_ref[...] = jnp.where(cls == xv, _HOT, _LOW)


def kernel(x):
    x3 = x.astype(jnp.int32).reshape(_GRID, 1, _BI)
    out_t = pl.pallas_call(
        _body,
        grid=(_GRID,),
        in_specs=[pl.BlockSpec((1, 1, _BI), lambda i: (i, 0, 0))],
        out_specs=pl.BlockSpec((_NB_CLASSES, _BI), lambda i: (0, i)),
        out_shape=jax.ShapeDtypeStruct((_NB_CLASSES, _N_ROWS), jnp.float32),
        compiler_params=pltpu.CompilerParams(
            dimension_semantics=("arbitrary",),
        ),
    )(x3)
    return out_t.T
